# baseline (speedup 1.0000x reference)
import functools

import jax
import jax.numpy as jnp
from jax import lax
from jax.experimental import pallas as pl
from jax.experimental.pallas import tpu as pltpu
from jax.experimental.pallas import tpu_sc as plsc

B = 16384
F = 26
V = 100000
D = 32

NC = 2
NS = 16
NW = NC * NS

BF = B * F
PER_W = BF // NW
CHUNK = 1024
SUB = 128
N_CHUNKS = PER_W // CHUNK
assert PER_W % CHUNK == 0 and CHUNK % SUB == 0


FD = F * D
VT_FULL = V // 128
V_TAIL = V - VT_FULL * 128
WV = 384
NCH_FULL = VT_FULL * 128 // WV
WV2 = VT_FULL * 128 - NCH_FULL * WV
N_UNITS = F * NCH_FULL


@functools.lru_cache(maxsize=1)
def _make_relayout():
    mesh = plsc.VectorSubcoreMesh(
        core_axis_name="c", subcore_axis_name="s", num_cores=NC, num_subcores=NS
    )

    NBUF = 4

    @functools.partial(
        pl.kernel,
        mesh=mesh,
        out_type=jax.ShapeDtypeStruct((F * V * D,), jnp.float32),
        scratch_types=[
            [pltpu.VMEM((D, WV), jnp.float32)] * NBUF,
            [pltpu.VMEM((WV * D,), jnp.float32)] * NBUF,
            [pltpu.SemaphoreType.DMA] * NBUF,
            [pltpu.SemaphoreType.DMA] * NBUF,
        ],
        compiler_params=pltpu.CompilerParams(
            use_tc_tiling_on_sc=True, needs_layout_passes=False
        ),
    )
    def relayout(src_hbm, tail_hbm, out_hbm, in_v, out_v, in_sems, out_sems):
        wid = lax.axis_index("s") * NC + lax.axis_index("c")
        lane = lax.iota(jnp.int32, 16)
        lane32 = lane * D
        n_mine = jnp.where(wid < N_UNITS % NW, N_UNITS // NW + 1, N_UNITS // NW)

        def transpose(b, width):
            dvecs = [(jnp.arange(16, dtype=jnp.int32) + d0) % D for d0 in range(D)]

            def vb_body(vb, carry):
                vv, vv32 = carry
                for d0 in range(0, D, 8):
                    vals = [
                        plsc.load_gather(in_v[b], [dvecs[d0 + k], vv])
                        for k in range(8)
                    ]
                    for k in range(8):
                        plsc.store_scatter(
                            out_v[b], [vv32 + dvecs[d0 + k]], vals[k]
                        )
                return vv + 16, vv32 + 16 * D

            lax.fori_loop(0, width // 16, vb_body, (lane, lane32))

        @pl.when(wid < F)
        def _ragged_end():
            v0 = NCH_FULL * WV
            pltpu.sync_copy(
                src_hbm.at[pl.ds(wid * D, D), pl.ds(v0, WV2)],
                in_v[0].at[:, pl.ds(0, WV2)],
            )
            transpose(0, WV2)
            pltpu.sync_copy(
                out_v[0].at[pl.ds(0, WV2 * D)],
                out_hbm.at[pl.ds((wid * V + v0) * D, WV2 * D)],
            )
            pltpu.sync_copy(
                tail_hbm.at[pl.ds(wid * (V_TAIL * D), V_TAIL * D)],
                out_v[0].at[pl.ds(0, V_TAIL * D)],
            )
            pltpu.sync_copy(
                out_v[0].at[pl.ds(0, V_TAIL * D)],
                out_hbm.at[pl.ds((wid * V + VT_FULL * 128) * D, V_TAIL * D)],
            )

        def in_slab(blk):
            f = blk // NCH_FULL
            v0 = (blk % NCH_FULL) * WV
            return src_hbm.at[pl.ds(f * D, D), pl.ds(v0, WV)]

        def out_run(blk):
            f = blk // NCH_FULL
            v0 = (blk % NCH_FULL) * WV
            return out_hbm.at[pl.ds((f * V + v0) * D, WV * D)]

        def group_body(g, carry):
            for b in range(NBUF):
                n = g * NBUF + b
                blk = n * NW + wid

                @pl.when(n < n_mine)
                def _blk(b=b, n=n, blk=blk):
                    pltpu.make_async_copy(
                        in_slab(blk), in_v[b], in_sems[b]
                    ).wait()

                    @pl.when(g > 0)
                    def _drain_out():
                        pltpu.make_async_copy(
                            out_v[b],
                            out_run(blk),
                            out_sems[b],
                        ).wait()

                    transpose(b, WV)
                    pltpu.async_copy(
                        out_v[b], out_run(blk), out_sems[b]
                    )

                    @pl.when(n + NBUF < n_mine)
                    def _next_in(b=b):
                        pltpu.async_copy(
                            in_slab((n + NBUF) * NW + wid),
                            in_v[b],
                            in_sems[b],
                        )

            return carry

        for b in range(NBUF):
            @pl.when(b < n_mine)
            def _prime(b=b):
                pltpu.async_copy(in_slab(b * NW + wid), in_v[b], in_sems[b])

        n_groups = (n_mine + NBUF - 1) // NBUF
        lax.fori_loop(0, n_groups, group_body, 0)

        for b in range(NBUF):
            @pl.when(b < n_mine)
            def _drain(b=b):
                pltpu.make_async_copy(
                    out_v[b],
                    out_hbm.at[pl.ds(0, WV * D)],
                    out_sems[b],
                ).wait()

    return relayout


@functools.lru_cache(maxsize=1)
def _make_gather():
    mesh = plsc.VectorSubcoreMesh(
        core_axis_name="c", subcore_axis_name="s", num_cores=NC, num_subcores=NS
    )

    @functools.partial(
        pl.kernel,
        mesh=mesh,
        out_type=jax.ShapeDtypeStruct((BF, D), jnp.float32),
        scratch_types=[
            [pltpu.VMEM((CHUNK,), jnp.int32)] * 2,
            [pltpu.VMEM((CHUNK, D), jnp.float32)] * 2,
            [pltpu.SemaphoreType.DMA] * 2,
            [pltpu.SemaphoreType.DMA] * 2,
            pltpu.SemaphoreType.DMA,
        ],
        compiler_params=pltpu.CompilerParams(use_tc_tiling_on_sc=False),
    )
    def gather_rows(table_hbm, gidx_hbm, out_hbm, idx_v, rows_v, isems, osems, gsem):
        wid = lax.axis_index("s") * NC + lax.axis_index("c")
        base = wid * PER_W

        def idx_slab(c):
            return gidx_hbm.at[pl.ds(base + c * CHUNK, CHUNK)]

        def out_slab(c):
            return out_hbm.at[pl.ds(base + c * CHUNK, CHUNK)]

        def do_chunk(c, b, drain_first, fire_next=True):
            pltpu.make_async_copy(idx_slab(c), idx_v[b], isems[b]).wait()

            if fire_next:
                pltpu.async_copy(idx_slab(c + 1), idx_v[1 - b], isems[1 - b])

            def _drain_out():
                pltpu.make_async_copy(
                    rows_v[b], out_slab(c), osems[b]
                ).wait()

            if isinstance(drain_first, bool):
                if drain_first:
                    _drain_out()
            else:
                pl.when(drain_first)(_drain_out)

            copies = [
                pltpu.async_copy(
                    table_hbm.at[idx_v[b].at[pl.ds(j * SUB, SUB)]],
                    rows_v[b].at[pl.ds(j * SUB, SUB)],
                    gsem,
                )
                for j in range(CHUNK // SUB)
            ]
            for cp in copies:
                cp.wait()
            pltpu.async_copy(rows_v[b], out_slab(c), osems[b])

        pltpu.async_copy(idx_slab(0), idx_v[0], isems[0])

        def group_body(g, carry):
            for b in range(2):
                do_chunk(g * 2 + b, b, g > 0)
            return carry

        lax.fori_loop(0, N_CHUNKS // 2, group_body, 0)
        do_chunk(N_CHUNKS - 1, (N_CHUNKS - 1) % 2, True, fire_next=False)

        for b in range(2):
            pltpu.make_async_copy(
                rows_v[b], out_hbm.at[pl.ds(0, CHUNK)], osems[b]
            ).wait()

    return gather_rows


def _mm_body(x_ref, w_ref, b_ref, o_ref):
    o_ref[...] = (
        jnp.dot(x_ref[...], w_ref[...], preferred_element_type=jnp.float32)
        + b_ref[...]
    )


_MM_BLK = 1024


def _project(x, wt, b2):
    return pl.pallas_call(
        _mm_body,
        grid=(B // _MM_BLK,),
        in_specs=[
            pl.BlockSpec((_MM_BLK, F * D), lambda i: (i, 0)),
            pl.BlockSpec((F * D, D), lambda i: (0, 0)),
            pl.BlockSpec((1, D), lambda i: (0, 0)),
        ],
        out_specs=pl.BlockSpec((_MM_BLK, D), lambda i: (i, 0)),
        out_shape=jax.ShapeDtypeStruct((B, D), jnp.float32),
    )(x, wt, b2)


def kernel(pars, tables, W, b):
    offs = (jnp.arange(F, dtype=jnp.int32) * V)[None, :]
    gidx = (pars.astype(jnp.int32) + offs).reshape(BF)
    tt2d = tables.transpose(0, 2, 1).reshape(FD, V)
    tail = tables[:, VT_FULL * 128 :, :].reshape(F * V_TAIL * D)
    packed = _make_relayout()(tt2d, tail)
    table2d = packed.reshape(F * V, D)
    rows = _make_gather()(table2d, gidx)
    x = rows.reshape(B, F * D)
    return _project(x, W.T, b.reshape(1, D))


# speedup vs baseline: 1.0394x; 1.0394x over previous
import functools

import jax
import jax.numpy as jnp
from jax import lax
from jax.experimental import pallas as pl
from jax.experimental.pallas import tpu as pltpu
from jax.experimental.pallas import tpu_sc as plsc

B = 16384
F = 26
V = 100000
D = 32

NC = 2
NS = 16
NW = NC * NS

BF = B * F
PER_W = BF // NW
CHUNK = 1024
SUB = 128
N_CHUNKS = PER_W // CHUNK
assert PER_W % CHUNK == 0 and CHUNK % SUB == 0


FD = F * D
VT_FULL = V // 128
V_TAIL = V - VT_FULL * 128
WV = 512
NCH_FULL = VT_FULL * 128 // WV
WV2 = VT_FULL * 128 - NCH_FULL * WV
N_UNITS = F * NCH_FULL


@functools.lru_cache(maxsize=1)
def _make_relayout():
    mesh = plsc.VectorSubcoreMesh(
        core_axis_name="c", subcore_axis_name="s", num_cores=NC, num_subcores=NS
    )

    NBUF = 3

    @functools.partial(
        pl.kernel,
        mesh=mesh,
        out_type=jax.ShapeDtypeStruct((F * V * D,), jnp.float32),
        scratch_types=[
            [pltpu.VMEM((D, WV), jnp.float32)] * NBUF,
            [pltpu.VMEM((WV * D,), jnp.float32)] * NBUF,
            [pltpu.SemaphoreType.DMA] * NBUF,
            [pltpu.SemaphoreType.DMA] * NBUF,
        ],
        compiler_params=pltpu.CompilerParams(
            use_tc_tiling_on_sc=True, needs_layout_passes=False
        ),
    )
    def relayout(src_hbm, tail_hbm, out_hbm, in_v, out_v, in_sems, out_sems):
        wid = lax.axis_index("s") * NC + lax.axis_index("c")
        lane = lax.iota(jnp.int32, 16)
        lane32 = lane * D
        n_mine = jnp.where(wid < N_UNITS % NW, N_UNITS // NW + 1, N_UNITS // NW)

        def transpose(b, width):
            dvecs = [(jnp.arange(16, dtype=jnp.int32) + d0) % D for d0 in range(D)]

            def vb_body(vb, carry):
                vv, vv32 = carry
                for d0 in range(0, D, 8):
                    vals = [
                        plsc.load_gather(in_v[b], [dvecs[d0 + k], vv])
                        for k in range(8)
                    ]
                    for k in range(8):
                        plsc.store_scatter(
                            out_v[b], [vv32 + dvecs[d0 + k]], vals[k]
                        )
                return vv + 16, vv32 + 16 * D

            lax.fori_loop(0, width // 16, vb_body, (lane, lane32))

        @pl.when(wid < F)
        def _ragged_end():
            v0 = NCH_FULL * WV
            pltpu.sync_copy(
                src_hbm.at[pl.ds(wid * D, D), pl.ds(v0, WV2)],
                in_v[0].at[:, pl.ds(0, WV2)],
            )
            transpose(0, WV2)
            pltpu.sync_copy(
                out_v[0].at[pl.ds(0, WV2 * D)],
                out_hbm.at[pl.ds((wid * V + v0) * D, WV2 * D)],
            )
            pltpu.sync_copy(
                tail_hbm.at[pl.ds(wid * (V_TAIL * D), V_TAIL * D)],
                out_v[0].at[pl.ds(0, V_TAIL * D)],
            )
            pltpu.sync_copy(
                out_v[0].at[pl.ds(0, V_TAIL * D)],
                out_hbm.at[pl.ds((wid * V + VT_FULL * 128) * D, V_TAIL * D)],
            )

        def in_slab(blk):
            f = blk // NCH_FULL
            v0 = (blk % NCH_FULL) * WV
            return src_hbm.at[pl.ds(f * D, D), pl.ds(v0, WV)]

        def out_run(blk):
            f = blk // NCH_FULL
            v0 = (blk % NCH_FULL) * WV
            return out_hbm.at[pl.ds((f * V + v0) * D, WV * D)]

        def group_body(g, carry):
            for b in range(NBUF):
                n = g * NBUF + b
                blk = n * NW + wid

                @pl.when(n < n_mine)
                def _blk(b=b, n=n, blk=blk):
                    pltpu.make_async_copy(
                        in_slab(blk), in_v[b], in_sems[b]
                    ).wait()

                    @pl.when(g > 0)
                    def _drain_out():
                        pltpu.make_async_copy(
                            out_v[b],
                            out_run(blk),
                            out_sems[b],
                        ).wait()

                    transpose(b, WV)
                    pltpu.async_copy(
                        out_v[b], out_run(blk), out_sems[b]
                    )

                    @pl.when(n + NBUF < n_mine)
                    def _next_in(b=b):
                        pltpu.async_copy(
                            in_slab((n + NBUF) * NW + wid),
                            in_v[b],
                            in_sems[b],
                        )

            return carry

        for b in range(NBUF):
            @pl.when(b < n_mine)
            def _prime(b=b):
                pltpu.async_copy(in_slab(b * NW + wid), in_v[b], in_sems[b])

        n_groups = (n_mine + NBUF - 1) // NBUF
        lax.fori_loop(0, n_groups, group_body, 0)

        for b in range(NBUF):
            @pl.when(b < n_mine)
            def _drain(b=b):
                pltpu.make_async_copy(
                    out_v[b],
                    out_hbm.at[pl.ds(0, WV * D)],
                    out_sems[b],
                ).wait()

    return relayout


@functools.lru_cache(maxsize=1)
def _make_gather():
    mesh = plsc.VectorSubcoreMesh(
        core_axis_name="c", subcore_axis_name="s", num_cores=NC, num_subcores=NS
    )

    @functools.partial(
        pl.kernel,
        mesh=mesh,
        out_type=jax.ShapeDtypeStruct((BF, D), jnp.float32),
        scratch_types=[
            [pltpu.VMEM((CHUNK,), jnp.int32)] * 2,
            [pltpu.VMEM((CHUNK, D), jnp.float32)] * 2,
            [pltpu.SemaphoreType.DMA] * 2,
            [pltpu.SemaphoreType.DMA] * 2,
            pltpu.SemaphoreType.DMA,
        ],
        compiler_params=pltpu.CompilerParams(use_tc_tiling_on_sc=False),
    )
    def gather_rows(table_hbm, gidx_hbm, out_hbm, idx_v, rows_v, isems, osems, gsem):
        wid = lax.axis_index("s") * NC + lax.axis_index("c")
        base = wid * PER_W

        def idx_slab(c):
            return gidx_hbm.at[pl.ds(base + c * CHUNK, CHUNK)]

        def out_slab(c):
            return out_hbm.at[pl.ds(base + c * CHUNK, CHUNK)]

        def do_chunk(c, b, drain_first, fire_next=True):
            pltpu.make_async_copy(idx_slab(c), idx_v[b], isems[b]).wait()

            if fire_next:
                pltpu.async_copy(idx_slab(c + 1), idx_v[1 - b], isems[1 - b])

            def _drain_out():
                pltpu.make_async_copy(
                    rows_v[b], out_slab(c), osems[b]
                ).wait()

            if isinstance(drain_first, bool):
                if drain_first:
                    _drain_out()
            else:
                pl.when(drain_first)(_drain_out)

            copies = [
                pltpu.async_copy(
                    table_hbm.at[idx_v[b].at[pl.ds(j * SUB, SUB)]],
                    rows_v[b].at[pl.ds(j * SUB, SUB)],
                    gsem,
                )
                for j in range(CHUNK // SUB)
            ]
            for cp in copies:
                cp.wait()
            pltpu.async_copy(rows_v[b], out_slab(c), osems[b])

        pltpu.async_copy(idx_slab(0), idx_v[0], isems[0])

        def group_body(g, carry):
            for b in range(2):
                do_chunk(g * 2 + b, b, g > 0)
            return carry

        lax.fori_loop(0, N_CHUNKS // 2, group_body, 0)
        do_chunk(N_CHUNKS - 1, (N_CHUNKS - 1) % 2, True, fire_next=False)

        for b in range(2):
            pltpu.make_async_copy(
                rows_v[b], out_hbm.at[pl.ds(0, CHUNK)], osems[b]
            ).wait()

    return gather_rows


def _mm_body(x_ref, w_ref, b_ref, o_ref):
    o_ref[...] = (
        jnp.dot(x_ref[...], w_ref[...], preferred_element_type=jnp.float32)
        + b_ref[...]
    )


_MM_BLK = 1024


def _project(x, wt, b2):
    return pl.pallas_call(
        _mm_body,
        grid=(B // _MM_BLK,),
        in_specs=[
            pl.BlockSpec((_MM_BLK, F * D), lambda i: (i, 0)),
            pl.BlockSpec((F * D, D), lambda i: (0, 0)),
            pl.BlockSpec((1, D), lambda i: (0, 0)),
        ],
        out_specs=pl.BlockSpec((_MM_BLK, D), lambda i: (i, 0)),
        out_shape=jax.ShapeDtypeStruct((B, D), jnp.float32),
    )(x, wt, b2)


def kernel(pars, tables, W, b):
    offs = (jnp.arange(F, dtype=jnp.int32) * V)[None, :]
    gidx = (pars.astype(jnp.int32) + offs).reshape(BF)
    tt2d = tables.transpose(0, 2, 1).reshape(FD, V)
    tail = tables[:, VT_FULL * 128 :, :].reshape(F * V_TAIL * D)
    packed = _make_relayout()(tt2d, tail)
    table2d = packed.reshape(F * V, D)
    rows = _make_gather()(table2d, gidx)
    x = rows.reshape(B, F * D)
    return _project(x, W.T, b.reshape(1, D))
